# Initial kernel scaffold; baseline (speedup 1.0000x reference)
import jax
import jax.numpy as jnp
from jax.experimental import pallas as pl


def kernel(inputs, emb0, emb1, emb2, emb3, proj0, proj1, proj2, proj3):
    raise NotImplementedError("write your pallas kernel here")


# speedup vs baseline: 8.2243x; 8.2243x over previous
import functools

import jax
import jax.numpy as jnp
from jax import lax
from jax.experimental import pallas as pl
from jax.experimental.pallas import tpu as pltpu
from jax.experimental.pallas import tpu_sc as plsc

D_MODEL = 128
CUT = [0, 20000, 100000, 500000, 1000000]
DS = [128, 32, 8, 2]
NC, NS = 2, 16
NW = NC * NS
NOG = D_MODEL // 16


def _iota16():
  return lax.iota(jnp.int32, 16)


def _splat(x):
  return jnp.full((16,), x, dtype=jnp.int32)


def _project_small(rows_v, pT_v, out_v, iota, d, colb=None):
  if colb is None:
    colb = jnp.zeros((16,), jnp.int32)
  cols = [plsc.load_gather(rows_v, [iota, colb + j]) for j in range(d)]
  s = [[cols[j][t] for j in range(d)] for t in range(16)]
  for og in range(NOG):
    pblk = [pT_v[j, pl.ds(og * 16, 16)] for j in range(d)]
    for t in range(16):
      acc = s[t][0] * pblk[0]
      for j in range(1, d):
        acc = acc + s[t][j] * pblk[j]
      out_v[t, pl.ds(og * 16, 16)] = acc


def _project_wide(rows_v, pT_v, out_v, iota, d):
  njb = d // 16
  zeros = jnp.zeros((16,), jnp.float32)

  def pbody(p, _):
    og, jb = p // njb, p % njb
    cols = [plsc.load_gather(rows_v, [iota, _splat(jb * 16 + jj)])
            for jj in range(16)]
    pblk = [pT_v[jb * 16 + jj, pl.ds(og * 16, 16)] for jj in range(16)]
    for t in range(16):
      prev = out_v[t, pl.ds(og * 16, 16)]
      acc = jnp.where(jb == 0, zeros, prev)
      for jj in range(16):
        acc = acc + cols[jj][t] * pblk[jj]
      out_v[t, pl.ds(og * 16, 16)] = acc
    return 0

  lax.fori_loop(0, NOG * njb, pbody, 0)


def _body(tok_hbm, e0, e1, e2, e3, p0, p1, p2, p3, out_hbm,
          tok_v, loc_l, pos_l, rows_vs, out_v, pT_vs, sem, sem2, T):
  embs = [e0, e1, e2, e3]
  pTs = [p0, p1, p2, p3]
  wid = lax.axis_index("s") * NC + lax.axis_index("c")
  base = wid * T

  pltpu.sync_copy(tok_hbm.at[pl.ds(base, T)], tok_v)
  for c in range(4):
    pltpu.sync_copy(pTs[c], pT_vs[c])

  iota = _iota16()

  for c in range(4):
    d = DS[c]
    start, end = CUT[c], CUT[c + 1]
    rows_v = rows_vs[c]
    pT_v = pT_vs[c]

    def cbody(i, count, start=start, end=end):
      v = tok_v[pl.ds(i * 16, 16)]
      m = (v >= start) & (v < end)
      incl = plsc.cumsum(m.astype(jnp.int32))
      dest = count + incl - 1
      plsc.store_scatter(loc_l, [dest], v - start, mask=m)
      plsc.store_scatter(pos_l, [dest], base + i * 16 + iota, mask=m)
      return count + incl[15]

    count = lax.fori_loop(0, T // 16, cbody, jnp.int32(0))

    @pl.when(count > 0)
    def _():
      lastloc = loc_l[pl.ds(count - 1, 16)][0]
      lastpos = pos_l[pl.ds(count - 1, 16)][0]
      plsc.store_scatter(loc_l, [_splat(count) + iota], _splat(lastloc))
      plsc.store_scatter(pos_l, [_splat(count) + iota], _splat(lastpos))

    nch = (count + 15) // 16

    def gbody(g, _, c=c, d=d, emb=embs[c], rows_v=rows_v, pT_v=pT_v):
      locv = loc_l[pl.ds(g * 16, 16)]
      posv = pos_l[pl.ds(g * 16, 16)]
      if c == 3:
        pltpu.async_copy(emb.at[locv // 4], rows_v, sem).wait()
        _project_small(rows_v, pT_v, out_v, iota, d, colb=(locv % 4) * 2)
      else:
        pltpu.async_copy(emb.at[locv], rows_v, sem).wait()
        if d <= 16:
          _project_small(rows_v, pT_v, out_v, iota, d)
        else:
          _project_wide(rows_v, pT_v, out_v, iota, d)
      pltpu.async_copy(out_v, out_hbm.at[posv], sem2).wait()
      return 0

    lax.fori_loop(0, nch, gbody, 0)


def kernel(inputs, emb0, emb1, emb2, emb3, proj0, proj1, proj2, proj3):
  n = inputs.shape[0] * inputs.shape[1]
  assert n % (NW * 16) == 0
  T = n // NW
  flat = inputs.reshape(n)
  scale = jnp.float32(D_MODEL ** 0.5)
  pTs = [jnp.transpose(p) * scale for p in (proj0, proj1, proj2, proj3)]
  emb3 = emb3.reshape(emb3.shape[0] // 4, 8)

  mesh = plsc.VectorSubcoreMesh(core_axis_name="c", subcore_axis_name="s",
                                num_cores=NC, num_subcores=NS)
  run = pl.kernel(
      functools.partial(_body, T=T),
      out_type=jax.ShapeDtypeStruct((n, D_MODEL), jnp.float32),
      mesh=mesh,
      compiler_params=pltpu.CompilerParams(use_tc_tiling_on_sc=False,
                                           needs_layout_passes=False),
      scratch_types=[
          pltpu.VMEM((T,), jnp.int32),
          pltpu.VMEM((T + 16,), jnp.int32),
          pltpu.VMEM((T + 16,), jnp.int32),
          [pltpu.VMEM((16, max(d, 8)), jnp.float32) for d in DS],
          pltpu.VMEM((16, D_MODEL), jnp.float32),
          [pltpu.VMEM((d, D_MODEL), jnp.float32) for d in DS],
          pltpu.SemaphoreType.DMA,
          pltpu.SemaphoreType.DMA,
      ],
  )
  out = run(flat, emb0, emb1, emb2, emb3, *pTs)
  return out.reshape(inputs.shape + (D_MODEL,))


# speedup vs baseline: 8.8927x; 1.0813x over previous
import functools

import jax
import jax.numpy as jnp
from jax import lax
from jax.experimental import pallas as pl
from jax.experimental.pallas import tpu as pltpu
from jax.experimental.pallas import tpu_sc as plsc

D_MODEL = 128
CUT = [0, 20000, 100000, 500000, 1000000]
DS = [128, 32, 8, 2]
NC, NS = 2, 16
NW = NC * NS
NOG = D_MODEL // 16


def _iota16():
  return lax.iota(jnp.int32, 16)


def _splat(x):
  return jnp.full((16,), x, dtype=jnp.int32)


def _project_small(rows_v, pT_v, out_v, iota, d, colb=None):
  if colb is None:
    colb = jnp.zeros((16,), jnp.int32)
  cols = [plsc.load_gather(rows_v, [iota, colb + j]) for j in range(d)]
  s = [[cols[j][t] for j in range(d)] for t in range(16)]
  for og in range(NOG):
    pblk = [pT_v[j, pl.ds(og * 16, 16)] for j in range(d)]
    for t in range(16):
      acc = s[t][0] * pblk[0]
      for j in range(1, d):
        acc = acc + s[t][j] * pblk[j]
      out_v[t, pl.ds(og * 16, 16)] = acc


def _project_wide(rows_v, pT_v, out_v, iota, d):
  njb = d // 16
  zeros = jnp.zeros((16,), jnp.float32)

  def pbody(p, _):
    og, jb = p // njb, p % njb
    cols = [plsc.load_gather(rows_v, [iota, _splat(jb * 16 + jj)])
            for jj in range(16)]
    pblk = [pT_v[jb * 16 + jj, pl.ds(og * 16, 16)] for jj in range(16)]
    for t in range(16):
      prev = out_v[t, pl.ds(og * 16, 16)]
      acc = jnp.where(jb == 0, zeros, prev)
      for jj in range(16):
        acc = acc + cols[jj][t] * pblk[jj]
      out_v[t, pl.ds(og * 16, 16)] = acc
    return 0

  lax.fori_loop(0, NOG * njb, pbody, 0)


def _gidx(locv, c):
  return locv // 4 if c == 3 else locv


def _body(tok_hbm, e0, e1, e2, e3, p0, p1, p2, p3, out_hbm,
          tok_v, loc_l, pos_l, rows_vs, out_vs, pT_vs, gsem, ssems, T):
  embs = [e0, e1, e2, e3]
  pTs = [p0, p1, p2, p3]
  wid = lax.axis_index("s") * NC + lax.axis_index("c")
  base = wid * T

  pltpu.sync_copy(tok_hbm.at[pl.ds(base, T)], tok_v)
  for c in range(4):
    pltpu.sync_copy(pTs[c], pT_vs[c])

  iota = _iota16()

  for c in range(4):
    d = DS[c]
    start, end = CUT[c], CUT[c + 1]
    rows_v = rows_vs[c]
    pT_v = pT_vs[c]

    def cbody(i, count, start=start, end=end):
      v = tok_v[pl.ds(i * 16, 16)]
      m = (v >= start) & (v < end)
      incl = plsc.cumsum(m.astype(jnp.int32))
      dest = count + incl - 1
      plsc.store_scatter(loc_l, [dest], v - start, mask=m)
      plsc.store_scatter(pos_l, [dest], base + i * 16 + iota, mask=m)
      return count + incl[15]

    count = lax.fori_loop(0, T // 16, cbody, jnp.int32(0))

    @pl.when(count > 0)
    def _(c=c, d=d, emb=embs[c], rows_v=rows_v, pT_v=pT_v, count=count):
      lastloc = loc_l[pl.ds(count - 1, 16)][0]
      lastpos = pos_l[pl.ds(count - 1, 16)][0]
      for k in range(3):
        plsc.store_scatter(loc_l, [_splat(count + k * 16) + iota],
                           _splat(lastloc))
        plsc.store_scatter(pos_l, [_splat(count + k * 16) + iota],
                           _splat(lastpos))
      nch = ((count + 31) // 32) * 2

      loc0 = loc_l[pl.ds(0, 16)]
      pltpu.async_copy(emb.at[_gidx(loc0, c)], rows_v[0], gsem)

      def g2body(g2, _):
        for ph in range(2):
          g = g2 * 2 + ph
          locv = loc_l[pl.ds(g * 16, 16)]
          posv = pos_l[pl.ds(g * 16, 16)]
          pltpu.make_async_copy(emb.at[_gidx(locv, c)], rows_v[ph],
                                gsem).wait()

          @pl.when(g + 1 < nch)
          def _():
            locn = loc_l[pl.ds((g + 1) * 16, 16)]
            pltpu.async_copy(emb.at[_gidx(locn, c)], rows_v[1 - ph], gsem)

          @pl.when(g >= 2)
          def _():
            pltpu.make_async_copy(out_vs[ph], out_hbm.at[posv],
                                  ssems[ph]).wait()

          if c == 3:
            _project_small(rows_v[ph], pT_v, out_vs[ph], iota, d,
                           colb=(locv % 4) * 2)
          elif d <= 16:
            _project_small(rows_v[ph], pT_v, out_vs[ph], iota, d)
          else:
            _project_wide(rows_v[ph], pT_v, out_vs[ph], iota, d)
          pltpu.async_copy(out_vs[ph], out_hbm.at[posv], ssems[ph])
        return 0

      lax.fori_loop(0, nch // 2, g2body, 0)
      pos0 = pos_l[pl.ds(0, 16)]
      for ph in range(2):
        pltpu.make_async_copy(out_vs[ph], out_hbm.at[pos0], ssems[ph]).wait()


def kernel(inputs, emb0, emb1, emb2, emb3, proj0, proj1, proj2, proj3):
  n = inputs.shape[0] * inputs.shape[1]
  assert n % (NW * 16) == 0
  T = n // NW
  flat = inputs.reshape(n)
  scale = jnp.float32(D_MODEL ** 0.5)
  pTs = [jnp.transpose(p) * scale for p in (proj0, proj1, proj2, proj3)]
  emb3 = emb3.reshape(emb3.shape[0] // 4, 8)

  mesh = plsc.VectorSubcoreMesh(core_axis_name="c", subcore_axis_name="s",
                                num_cores=NC, num_subcores=NS)
  run = pl.kernel(
      functools.partial(_body, T=T),
      out_type=jax.ShapeDtypeStruct((n, D_MODEL), jnp.float32),
      mesh=mesh,
      compiler_params=pltpu.CompilerParams(use_tc_tiling_on_sc=False,
                                           needs_layout_passes=False),
      scratch_types=[
          pltpu.VMEM((T,), jnp.int32),
          pltpu.VMEM((T + 48,), jnp.int32),
          pltpu.VMEM((T + 48,), jnp.int32),
          [[pltpu.VMEM((16, max(d, 8)), jnp.float32) for _ in range(2)]
           for d in DS],
          [pltpu.VMEM((16, D_MODEL), jnp.float32) for _ in range(2)],
          [pltpu.VMEM((d, D_MODEL), jnp.float32) for d in DS],
          pltpu.SemaphoreType.DMA,
          [pltpu.SemaphoreType.DMA for _ in range(2)],
      ],
  )
  out = run(flat, emb0, emb1, emb2, emb3, *pTs)
  return out.reshape(inputs.shape + (D_MODEL,))
